# baseline (speedup 1.0000x reference)
import functools

import jax
import jax.numpy as jnp
from jax import lax
from jax.experimental import pallas as pl
from jax.experimental.pallas import tpu as pltpu
from jax.experimental.pallas import tpu_sc as plsc

N_NODES = 10000
D_FEAT = 128
K = 10
ALPHA = 0.1
DROPOUT_P = 0.5

NPAD = 10240
DH = D_FEAT // 2
NT = 16
CHUNK = 128
NCH = 80
EPT = NCH * CHUNK
CAP = NT * EPT
RPT = NPAD // NT
UCH = RPT // CHUNK


def _appnp_body(col_hbm, colp_hbm, row_hbm, val_hbm, x_hbm, h_hbm,
                col_t, row_t, val_t, rows_v, a_v, b_v, zz_v, sem, agg_sp):
    c = lax.axis_index("c")
    s = lax.axis_index("s")

    def _zz(i, _):
        zz_v[i >> 2, pl.ds((i & 3) * 16, 16)] = jnp.zeros((16,), jnp.float32)
        return 0
    lax.fori_loop(0, CHUNK * 4, _zz, 0)

    def _init(u, _):
        roff = c * NPAD + s * RPT + u * CHUNK
        pltpu.sync_copy(x_hbm.at[pl.ds(roff, CHUNK)], a_v)
        pltpu.sync_copy(a_v, h_hbm.at[pl.ds(roff, CHUNK)])
        return 0
    lax.fori_loop(0, UCH, _init, 0)
    plsc.subcore_barrier()

    def _round(k, _):
        def _zero(u, _):
            pltpu.sync_copy(zz_v, agg_sp.at[pl.ds(s * RPT + u * CHUNK, CHUNK)])
            return 0
        with jax.named_scope("ph_zero"):
            lax.fori_loop(0, UCH, _zero, 0)

        blk = k * NT + s

        @pl.when(c == 0)
        def _():
            pltpu.sync_copy(col_hbm.at[blk], col_t)

        @pl.when(c != 0)
        def _():
            pltpu.sync_copy(colp_hbm.at[blk], col_t)

        with jax.named_scope("ph_stage"):
            pltpu.sync_copy(row_hbm.at[blk], row_t)
            pltpu.sync_copy(val_hbm.at[blk], val_t)

        plsc.subcore_barrier()

        def _chunk(j, _):
            pltpu.async_copy(h_hbm.at[col_t.at[j]], rows_v, sem).wait()

            def _scale(e, _):
                vs = plsc.load_gather(
                    val_t, [jnp.full((16,), j * CHUNK + e, jnp.int32)])
                for g in range(DH // 16):
                    sl = pl.ds(g * 16, 16)
                    rows_v[e, sl] = rows_v[e, sl] * vs
                return 0
            lax.fori_loop(0, CHUNK, _scale, 0)

            pltpu.sync_copy(rows_v, agg_sp.at[row_t.at[j]], add=True)
            return 0
        with jax.named_scope("ph_edges"):
            lax.fori_loop(0, NCH, _chunk, 0)
        plsc.subcore_barrier()

        def _update(u, _):
            roff = s * RPT + u * CHUNK
            hoff = c * NPAD + roff
            pltpu.sync_copy(agg_sp.at[pl.ds(roff, CHUNK)], a_v)
            pltpu.sync_copy(x_hbm.at[pl.ds(hoff, CHUNK)], b_v)

            def _fma(i, _):
                r = i >> 2
                sl = pl.ds((i & 3) * 16, 16)
                a_v[r, sl] = a_v[r, sl] * (1.0 - ALPHA) + b_v[r, sl] * ALPHA
                return 0
            lax.fori_loop(0, CHUNK * 4, _fma, 0)
            pltpu.sync_copy(a_v, h_hbm.at[pl.ds(hoff, CHUNK)])
            return 0
        with jax.named_scope("ph_upd"):
            lax.fori_loop(0, UCH, _update, 0)
        plsc.subcore_barrier()
        return 0

    lax.fori_loop(0, K, _round, 0)


@functools.partial(
    pl.kernel,
    out_type=jax.ShapeDtypeStruct((2 * NPAD, DH), jnp.float32),
    mesh=plsc.VectorSubcoreMesh(core_axis_name="c", subcore_axis_name="s"),
    compiler_params=pltpu.CompilerParams(needs_layout_passes=False,
                                         use_tc_tiling_on_sc=False),
    scratch_types=[
        pltpu.VMEM((NCH, CHUNK), jnp.int32),
        pltpu.VMEM((NCH, CHUNK), jnp.int32),
        pltpu.VMEM((EPT,), jnp.float32),
        pltpu.VMEM((CHUNK, DH), jnp.float32),
        pltpu.VMEM((CHUNK, DH), jnp.float32),
        pltpu.VMEM((CHUNK, DH), jnp.float32),
        pltpu.VMEM((CHUNK, DH), jnp.float32),
        pltpu.SemaphoreType.DMA,
        pltpu.VMEM_SHARED((NPAD, DH), jnp.float32),
    ],
)
def _appnp_sc(col_hbm, colp_hbm, row_hbm, val_hbm, x_hbm, h_hbm,
              col_t, row_t, val_t, rows_v, a_v, b_v, zz_v, sem, agg_sp):
    _appnp_body(col_hbm, colp_hbm, row_hbm, val_hbm, x_hbm, h_hbm,
                col_t, row_t, val_t, rows_v, a_v, b_v, zz_v, sem, agg_sp)


N_EDGES_FIXED = 320000


def _threefry2x32_np(k1, k2, x0, x1):
    import numpy as np
    r0 = (13, 15, 26, 6)
    r1 = (17, 29, 16, 24)
    ks0 = np.uint32(k1)
    ks1 = np.uint32(k2)
    ks2 = ks0 ^ ks1 ^ np.uint32(0x1BD11BDA)
    with np.errstate(over="ignore"):
        x0 = (x0 + ks0).astype(np.uint32)
        x1 = (x1 + ks1).astype(np.uint32)

        def rounds(x0, x1, rots):
            for r in rots:
                x0 = (x0 + x1).astype(np.uint32)
                x1 = ((x1 << np.uint32(r))
                      | (x1 >> np.uint32(32 - r))).astype(np.uint32)
                x1 = x1 ^ x0
            return x0, x1

        for rots, a0, a1, ctr in ((r0, ks1, ks2, 1), (r1, ks2, ks0, 2),
                                  (r0, ks0, ks1, 3), (r1, ks1, ks2, 4),
                                  (r0, ks2, ks0, 5)):
            x0, x1 = rounds(x0, x1, rots)
            x0 = (x0 + a0).astype(np.uint32)
            x1 = (x1 + a1 + np.uint32(ctr)).astype(np.uint32)
    return x0, x1


def _build_edge_constants():
    import numpy as np
    k1, k2 = np.uint32(0), np.uint32(42)
    idx_rounds = []
    valid_rounds = []
    for i in range(K):
        f1, f2 = _threefry2x32_np(k1, k2, np.uint32(0), np.uint32(i))
        hi = np.zeros(N_EDGES_FIXED, np.uint32)
        lo = np.arange(N_EDGES_FIXED, dtype=np.uint32)
        b1, b2 = _threefry2x32_np(f1, f2, hi, lo)
        bits = b1 ^ b2
        fb = (bits >> np.uint32(9)) | np.uint32(0x3F800000)
        floats = fb.view(np.float32) - np.float32(1.0)
        mask = floats < np.float32(1.0 - DROPOUT_P)
        keep = np.nonzero(mask)[0].astype(np.int32)
        cnt = keep.shape[0]
        assert cnt <= CAP
        idx = np.zeros(CAP, np.int32)
        idx[:cnt] = keep
        valid = np.arange(CAP) < cnt
        idx_rounds.append(idx)
        valid_rounds.append(valid)
    return np.stack(idx_rounds), np.stack(valid_rounds)


_IDX_ALL, _VALID_ALL = _build_edge_constants()


def kernel(x, edge_index, edge_weight):
    row = edge_index[0]
    col = edge_index[1]

    idx_all = _IDX_ALL
    valid_all = _VALID_ALL

    scale = 1.0 / (1.0 - DROPOUT_P)
    val_all = jnp.where(valid_all, edge_weight[idx_all] * scale, 0.0)
    col_all = jnp.where(valid_all, col[idx_all], 0).astype(jnp.int32)
    row_all = jnp.where(valid_all, row[idx_all], 0).astype(jnp.int32)

    xpad = jnp.pad(x, ((0, NPAD - N_NODES), (0, 0)))
    x_flat = jnp.concatenate([xpad[:, :DH], xpad[:, DH:]], axis=0)

    eshape = (K * NT, NCH, CHUNK)
    h = _appnp_sc(col_all.reshape(eshape), (col_all + NPAD).reshape(eshape),
                  row_all.reshape(eshape), val_all.reshape(K * NT, EPT),
                  x_flat)
    return jnp.concatenate([h[:N_NODES], h[NPAD:NPAD + N_NODES]], axis=1)


# speedup vs baseline: 18.4508x; 18.4508x over previous
import functools

import jax
import jax.numpy as jnp
from jax import lax
from jax.experimental import pallas as pl
from jax.experimental.pallas import tpu as pltpu
from jax.experimental.pallas import tpu_sc as plsc

N_NODES = 10000
D_FEAT = 128
K = 10
ALPHA = 0.1
DROPOUT_P = 0.5

NPAD = 10240
DH = D_FEAT // 2
NT = 16
CHUNK = 128
NCH = 160
EPT = NCH * CHUNK
E_PAD = NT * EPT
RPT = NPAD // NT
UCH = RPT // CHUNK


def _appnp_body(col_hbm, colp_hbm, row_hbm, val_hbm, x_hbm, h_hbm,
                col_t, row_t, val_t, rows_v, a_v, b_v, sem, agg_sp):
    c = lax.axis_index("c")
    s = lax.axis_index("s")

    def _init(u, _):
        roff = c * NPAD + s * RPT + u * CHUNK
        pltpu.sync_copy(x_hbm.at[pl.ds(roff, CHUNK)], a_v)
        pltpu.sync_copy(a_v, h_hbm.at[pl.ds(roff, CHUNK)])
        return 0
    lax.fori_loop(0, UCH, _init, 0)

    @pl.when(c == 0)
    def _():
        pltpu.sync_copy(col_hbm.at[s], col_t)

    @pl.when(c != 0)
    def _():
        pltpu.sync_copy(colp_hbm.at[s], col_t)

    pltpu.sync_copy(row_hbm.at[s], row_t)
    plsc.subcore_barrier()

    def _round(k, _):
        def _zb(i, _):
            b_v[i >> 2, pl.ds((i & 3) * 16, 16)] = jnp.zeros((16,), jnp.float32)
            return 0

        def _zero(u, _):
            pltpu.sync_copy(b_v, agg_sp.at[pl.ds(s * RPT + u * CHUNK, CHUNK)])
            return 0
        with jax.named_scope("ph_zero"):
            lax.fori_loop(0, CHUNK * 4, _zb, 0)
            lax.fori_loop(0, UCH, _zero, 0)

        with jax.named_scope("ph_stage"):
            pltpu.sync_copy(val_hbm.at[k * NT + s], val_t)
        plsc.subcore_barrier()

        def _chunk(j, _):
            pltpu.async_copy(h_hbm.at[col_t.at[j]], rows_v, sem).wait()

            def _scale(e, _):
                vs = plsc.load_gather(
                    val_t, [jnp.full((16,), j * CHUNK + e, jnp.int32)])
                for g in range(DH // 16):
                    sl = pl.ds(g * 16, 16)
                    rows_v[e, sl] = rows_v[e, sl] * vs
                return 0
            lax.fori_loop(0, CHUNK, _scale, 0)

            pltpu.sync_copy(rows_v, agg_sp.at[row_t.at[j]], add=True)
            return 0
        with jax.named_scope("ph_edges"):
            lax.fori_loop(0, NCH, _chunk, 0)
        plsc.subcore_barrier()

        def _update(u, _):
            roff = s * RPT + u * CHUNK
            hoff = c * NPAD + roff
            pltpu.sync_copy(agg_sp.at[pl.ds(roff, CHUNK)], a_v)
            pltpu.sync_copy(x_hbm.at[pl.ds(hoff, CHUNK)], b_v)

            def _fma(i, _):
                r = i >> 2
                sl = pl.ds((i & 3) * 16, 16)
                a_v[r, sl] = a_v[r, sl] * (1.0 - ALPHA) + b_v[r, sl] * ALPHA
                return 0
            lax.fori_loop(0, CHUNK * 4, _fma, 0)
            pltpu.sync_copy(a_v, h_hbm.at[pl.ds(hoff, CHUNK)])
            return 0
        with jax.named_scope("ph_upd"):
            lax.fori_loop(0, UCH, _update, 0)
        plsc.subcore_barrier()
        return 0

    lax.fori_loop(0, K, _round, 0)


@functools.partial(
    pl.kernel,
    out_type=jax.ShapeDtypeStruct((2 * NPAD, DH), jnp.float32),
    mesh=plsc.VectorSubcoreMesh(core_axis_name="c", subcore_axis_name="s"),
    compiler_params=pltpu.CompilerParams(needs_layout_passes=False,
                                         use_tc_tiling_on_sc=False),
    scratch_types=[
        pltpu.VMEM((NCH, CHUNK), jnp.int32),
        pltpu.VMEM((NCH, CHUNK), jnp.int32),
        pltpu.VMEM((EPT,), jnp.float32),
        pltpu.VMEM((CHUNK, DH), jnp.float32),
        pltpu.VMEM((CHUNK, DH), jnp.float32),
        pltpu.VMEM((CHUNK, DH), jnp.float32),
        pltpu.SemaphoreType.DMA,
        pltpu.VMEM_SHARED((NPAD, DH), jnp.float32),
    ],
)
def _appnp_sc(col_hbm, colp_hbm, row_hbm, val_hbm, x_hbm, h_hbm,
              col_t, row_t, val_t, rows_v, a_v, b_v, sem, agg_sp):
    _appnp_body(col_hbm, colp_hbm, row_hbm, val_hbm, x_hbm, h_hbm,
                col_t, row_t, val_t, rows_v, a_v, b_v, sem, agg_sp)


N_EDGES_FIXED = 320000


def _threefry2x32_np(k1, k2, x0, x1):
    import numpy as np
    r0 = (13, 15, 26, 6)
    r1 = (17, 29, 16, 24)
    ks0 = np.uint32(k1)
    ks1 = np.uint32(k2)
    ks2 = ks0 ^ ks1 ^ np.uint32(0x1BD11BDA)
    with np.errstate(over="ignore"):
        x0 = (x0 + ks0).astype(np.uint32)
        x1 = (x1 + ks1).astype(np.uint32)

        def rounds(x0, x1, rots):
            for r in rots:
                x0 = (x0 + x1).astype(np.uint32)
                x1 = ((x1 << np.uint32(r))
                      | (x1 >> np.uint32(32 - r))).astype(np.uint32)
                x1 = x1 ^ x0
            return x0, x1

        for rots, a0, a1, ctr in ((r0, ks1, ks2, 1), (r1, ks2, ks0, 2),
                                  (r0, ks0, ks1, 3), (r1, ks1, ks2, 4),
                                  (r0, ks2, ks0, 5)):
            x0, x1 = rounds(x0, x1, rots)
            x0 = (x0 + a0).astype(np.uint32)
            x1 = (x1 + a1 + np.uint32(ctr)).astype(np.uint32)
    return x0, x1


def _build_mask_constants():
    import numpy as np
    k1, k2 = np.uint32(0), np.uint32(42)
    masks = []
    for i in range(K):
        f1, f2 = _threefry2x32_np(k1, k2, np.uint32(0), np.uint32(i))
        hi = np.zeros(N_EDGES_FIXED, np.uint32)
        lo = np.arange(N_EDGES_FIXED, dtype=np.uint32)
        b1, b2 = _threefry2x32_np(f1, f2, hi, lo)
        bits = b1 ^ b2
        fb = (bits >> np.uint32(9)) | np.uint32(0x3F800000)
        floats = fb.view(np.float32) - np.float32(1.0)
        masks.append(floats < np.float32(1.0 - DROPOUT_P))
    return np.stack(masks)


_MASKS = _build_mask_constants()


def kernel(x, edge_index, edge_weight):
    row = edge_index[0]
    col = edge_index[1]
    n_edges = edge_weight.shape[0]
    epad = E_PAD - n_edges

    scale = 1.0 / (1.0 - DROPOUT_P)
    masks = jnp.asarray(_MASKS)
    val_all = jnp.where(masks, edge_weight * scale, 0.0)
    val_all = jnp.pad(val_all, ((0, 0), (0, epad)))
    col_p = jnp.pad(col.astype(jnp.int32), (0, epad))
    row_p = jnp.pad(row.astype(jnp.int32), (0, epad))

    xpad = jnp.pad(x, ((0, NPAD - N_NODES), (0, 0)))
    x_flat = jnp.concatenate([xpad[:, :DH], xpad[:, DH:]], axis=0)

    eshape = (NT, NCH, CHUNK)
    h = _appnp_sc(col_p.reshape(eshape), (col_p + NPAD).reshape(eshape),
                  row_p.reshape(eshape), val_all.reshape(K * NT, EPT),
                  x_flat)
    return jnp.concatenate([h[:N_NODES], h[NPAD:NPAD + N_NODES]], axis=1)


# speedup vs baseline: 32.2113x; 1.7458x over previous
import functools

import jax
import jax.numpy as jnp
from jax import lax
from jax.experimental import pallas as pl
from jax.experimental.pallas import tpu as pltpu
from jax.experimental.pallas import tpu_sc as plsc

N_NODES = 10000
D_FEAT = 128
K = 10
ALPHA = 0.1
DROPOUT_P = 0.5

NPAD = 10240
DH = D_FEAT // 2
NT = 16
CHUNK = 128
NCH = 160
NP2 = NCH // 2
EPT = NCH * CHUNK
E_PAD = NT * EPT
RPT = NPAD // NT
UCH = RPT // CHUNK


def _appnp_body(pk_hbm, val_hbm, x_hbm, h_hbm,
                pk_t, val_t, r0, r1, cc0, rc0, cc1, rc1, a_v, b_v,
                gs0, gs1, agg_sp):
    c = lax.axis_index("c")
    s = lax.axis_index("s")
    cbase = jnp.full((16,), c * NPAD, jnp.int32)

    def _init(u, _):
        roff = c * NPAD + s * RPT + u * CHUNK
        pltpu.sync_copy(x_hbm.at[pl.ds(roff, CHUNK)], a_v)
        pltpu.sync_copy(a_v, h_hbm.at[pl.ds(roff, CHUNK)])
        return 0
    lax.fori_loop(0, UCH, _init, 0)

    pltpu.sync_copy(pk_hbm.at[s], pk_t)
    plsc.subcore_barrier()

    def _unpack(j, cc, rc):
        for g in range(CHUNK // 16):
            sl = pl.ds(g * 16, 16)
            pk = pk_t[j, sl]
            cc[sl] = (pk & 0xFFFF) + cbase
            rc[sl] = pk >> 16

    def _scale(buf, joff):
        @plsc.parallel_loop(0, CHUNK, 1, unroll=4)
        def _(e):
            vs = plsc.load_gather(
                val_t, [jnp.full((16,), joff + e, jnp.int32)])
            for g in range(DH // 16):
                sl = pl.ds(g * 16, 16)
                buf[e, sl] = buf[e, sl] * vs

    def _round(k, _):
        def _zb(i, _):
            b_v[i >> 2, pl.ds((i & 3) * 16, 16)] = jnp.zeros((16,), jnp.float32)
            return 0

        def _zero(u, _):
            pltpu.sync_copy(b_v, agg_sp.at[pl.ds(s * RPT + u * CHUNK, CHUNK)])
            return 0
        with jax.named_scope("ph_zero"):
            lax.fori_loop(0, CHUNK * 4, _zb, 0)
            lax.fori_loop(0, UCH, _zero, 0)

        with jax.named_scope("ph_stage"):
            pltpu.sync_copy(val_hbm.at[k * NT + s], val_t)
        plsc.subcore_barrier()

        _unpack(0, cc0, rc0)
        pltpu.async_copy(h_hbm.at[cc0], r0, gs0)

        def _pair(p, _):
            j0 = p * 2
            _unpack(j0 + 1, cc1, rc1)
            pltpu.async_copy(h_hbm.at[cc1], r1, gs1)
            pltpu.make_async_copy(h_hbm.at[cc0], r0, gs0).wait()
            _scale(r0, j0 * CHUNK)
            pltpu.sync_copy(r0, agg_sp.at[rc0], add=True)

            @pl.when(p < NP2 - 1)
            def _():
                _unpack(j0 + 2, cc0, rc0)
                pltpu.async_copy(h_hbm.at[cc0], r0, gs0)

            pltpu.make_async_copy(h_hbm.at[cc1], r1, gs1).wait()
            _scale(r1, (j0 + 1) * CHUNK)
            pltpu.sync_copy(r1, agg_sp.at[rc1], add=True)
            return 0
        with jax.named_scope("ph_edges"):
            lax.fori_loop(0, NP2, _pair, 0)
        plsc.subcore_barrier()

        def _update(u, _):
            roff = s * RPT + u * CHUNK
            hoff = c * NPAD + roff
            pltpu.sync_copy(agg_sp.at[pl.ds(roff, CHUNK)], a_v)
            pltpu.sync_copy(x_hbm.at[pl.ds(hoff, CHUNK)], b_v)

            def _fma(i, _):
                r = i >> 2
                sl = pl.ds((i & 3) * 16, 16)
                a_v[r, sl] = a_v[r, sl] * (1.0 - ALPHA) + b_v[r, sl] * ALPHA
                return 0
            lax.fori_loop(0, CHUNK * 4, _fma, 0)
            pltpu.sync_copy(a_v, h_hbm.at[pl.ds(hoff, CHUNK)])
            return 0
        with jax.named_scope("ph_upd"):
            lax.fori_loop(0, UCH, _update, 0)
        plsc.subcore_barrier()
        return 0

    lax.fori_loop(0, K, _round, 0)


@functools.partial(
    pl.kernel,
    out_type=jax.ShapeDtypeStruct((2 * NPAD, DH), jnp.float32),
    mesh=plsc.VectorSubcoreMesh(core_axis_name="c", subcore_axis_name="s"),
    compiler_params=pltpu.CompilerParams(needs_layout_passes=False,
                                         use_tc_tiling_on_sc=False),
    scratch_types=[
        pltpu.VMEM((NCH, CHUNK), jnp.int32),
        pltpu.VMEM((EPT,), jnp.float32),
        pltpu.VMEM((CHUNK, DH), jnp.float32),
        pltpu.VMEM((CHUNK, DH), jnp.float32),
        pltpu.VMEM((CHUNK,), jnp.int32),
        pltpu.VMEM((CHUNK,), jnp.int32),
        pltpu.VMEM((CHUNK,), jnp.int32),
        pltpu.VMEM((CHUNK,), jnp.int32),
        pltpu.VMEM((CHUNK, DH), jnp.float32),
        pltpu.VMEM((CHUNK, DH), jnp.float32),
        pltpu.SemaphoreType.DMA,
        pltpu.SemaphoreType.DMA,
        pltpu.VMEM_SHARED((NPAD, DH), jnp.float32),
    ],
)
def _appnp_sc(pk_hbm, val_hbm, x_hbm, h_hbm,
              pk_t, val_t, r0, r1, cc0, rc0, cc1, rc1, a_v, b_v,
              gs0, gs1, agg_sp):
    _appnp_body(pk_hbm, val_hbm, x_hbm, h_hbm,
                pk_t, val_t, r0, r1, cc0, rc0, cc1, rc1, a_v, b_v,
                gs0, gs1, agg_sp)


N_EDGES_FIXED = 320000


def _threefry2x32_np(k1, k2, x0, x1):
    import numpy as np
    r0 = (13, 15, 26, 6)
    r1 = (17, 29, 16, 24)
    ks0 = np.uint32(k1)
    ks1 = np.uint32(k2)
    ks2 = ks0 ^ ks1 ^ np.uint32(0x1BD11BDA)
    with np.errstate(over="ignore"):
        x0 = (x0 + ks0).astype(np.uint32)
        x1 = (x1 + ks1).astype(np.uint32)

        def rounds(x0, x1, rots):
            for r in rots:
                x0 = (x0 + x1).astype(np.uint32)
                x1 = ((x1 << np.uint32(r))
                      | (x1 >> np.uint32(32 - r))).astype(np.uint32)
                x1 = x1 ^ x0
            return x0, x1

        for rots, a0, a1, ctr in ((r0, ks1, ks2, 1), (r1, ks2, ks0, 2),
                                  (r0, ks0, ks1, 3), (r1, ks1, ks2, 4),
                                  (r0, ks2, ks0, 5)):
            x0, x1 = rounds(x0, x1, rots)
            x0 = (x0 + a0).astype(np.uint32)
            x1 = (x1 + a1 + np.uint32(ctr)).astype(np.uint32)
    return x0, x1


def _build_mask_constants():
    import numpy as np
    k1, k2 = np.uint32(0), np.uint32(42)
    masks = []
    for i in range(K):
        f1, f2 = _threefry2x32_np(k1, k2, np.uint32(0), np.uint32(i))
        hi = np.zeros(N_EDGES_FIXED, np.uint32)
        lo = np.arange(N_EDGES_FIXED, dtype=np.uint32)
        b1, b2 = _threefry2x32_np(f1, f2, hi, lo)
        bits = b1 ^ b2
        fb = (bits >> np.uint32(9)) | np.uint32(0x3F800000)
        floats = fb.view(np.float32) - np.float32(1.0)
        masks.append(floats < np.float32(1.0 - DROPOUT_P))
    return np.stack(masks)


_MASKS = _build_mask_constants()


def kernel(x, edge_index, edge_weight):
    row = edge_index[0]
    col = edge_index[1]
    n_edges = edge_weight.shape[0]
    epad = E_PAD - n_edges

    scale = 1.0 / (1.0 - DROPOUT_P)
    masks = jnp.asarray(_MASKS)
    val_all = jnp.where(masks, edge_weight * scale, 0.0)
    val_all = jnp.pad(val_all, ((0, 0), (0, epad)))
    packed = (row.astype(jnp.int32) << 16) | col.astype(jnp.int32)
    packed = jnp.pad(packed, (0, epad))

    xpad = jnp.pad(x, ((0, NPAD - N_NODES), (0, 0)))
    x_flat = jnp.concatenate([xpad[:, :DH], xpad[:, DH:]], axis=0)

    h = _appnp_sc(packed.reshape(NT, NCH, CHUNK),
                  val_all.reshape(K * NT, EPT), x_flat)
    return jnp.concatenate([h[:N_NODES], h[NPAD:NPAD + N_NODES]], axis=1)


# speedup vs baseline: 63.1651x; 1.9610x over previous
import functools

import jax
import jax.numpy as jnp
from jax import lax
from jax.experimental import pallas as pl
from jax.experimental.pallas import tpu as pltpu
from jax.experimental.pallas import tpu_sc as plsc

N_NODES = 10000
D_FEAT = 128
K = 10
ALPHA = 0.1
DROPOUT_P = 0.5

NPAD = 10240
DH = D_FEAT // 2
NT = 16
CHUNK = 128
NCH = 160
NP2 = NCH // 2
EPT = NCH * CHUNK
E_PAD = NT * EPT
RPT = NPAD // NT
UCH = RPT // CHUNK


def _appnp_body(pk_hbm, val_hbm, x_hbm, h_hbm,
                pk_t, val_t, r0, r1, cc0, rc0, cc1, rc1, a_v, b_v,
                gs0, gs1, agg_sp):
    c = lax.axis_index("c")
    s = lax.axis_index("s")
    cbase = jnp.full((16,), c * NPAD, jnp.int32)

    def _init(u, _):
        roff = c * NPAD + s * RPT + u * CHUNK
        pltpu.sync_copy(x_hbm.at[pl.ds(roff, CHUNK)], a_v)
        pltpu.sync_copy(a_v, h_hbm.at[pl.ds(roff, CHUNK)])
        return 0
    lax.fori_loop(0, UCH, _init, 0)

    pltpu.sync_copy(pk_hbm.at[s], pk_t)
    plsc.subcore_barrier()

    def _unpack(j, cc, rc):
        for g in range(CHUNK // 16):
            sl = pl.ds(g * 16, 16)
            pk = pk_t[j, sl]
            cc[sl] = (pk & 0xFFFF) + cbase
            rc[sl] = pk >> 16

    def _scale(buf, joff):
        @plsc.parallel_loop(0, CHUNK, 1, unroll=4)
        def _(e):
            vs = plsc.load_gather(
                val_t, [jnp.full((16,), joff + e, jnp.int32)])
            for g in range(DH // 16):
                sl = pl.ds(g * 16, 16)
                buf[e, sl] = buf[e, sl] * vs

    def _round(k, _):
        def _zb(i, _):
            b_v[i >> 2, pl.ds((i & 3) * 16, 16)] = jnp.zeros((16,), jnp.float32)
            return 0

        def _zero(u, _):
            pltpu.sync_copy(b_v, agg_sp.at[pl.ds(s * RPT + u * CHUNK, CHUNK)])
            return 0
        with jax.named_scope("ph_zero"):
            lax.fori_loop(0, CHUNK * 4, _zb, 0)
            lax.fori_loop(0, UCH, _zero, 0)

        with jax.named_scope("ph_stage"):
            pltpu.sync_copy(val_hbm.at[k * NT + s], val_t)
        plsc.subcore_barrier()

        _unpack(0, cc0, rc0)
        pltpu.async_copy(h_hbm.at[cc0], r0, gs0)

        def _pair(p, _):
            j0 = p * 2
            _unpack(j0 + 1, cc1, rc1)
            pltpu.async_copy(h_hbm.at[cc1], r1, gs1)
            pltpu.make_async_copy(h_hbm.at[cc0], r0, gs0).wait()
            _scale(r0, j0 * CHUNK)
            pltpu.sync_copy(r0, agg_sp.at[rc0], add=True)

            @pl.when(p < NP2 - 1)
            def _():
                _unpack(j0 + 2, cc0, rc0)
                pltpu.async_copy(h_hbm.at[cc0], r0, gs0)

            pltpu.make_async_copy(h_hbm.at[cc1], r1, gs1).wait()
            _scale(r1, (j0 + 1) * CHUNK)
            pltpu.sync_copy(r1, agg_sp.at[rc1], add=True)
            return 0
        with jax.named_scope("ph_edges"):
            lax.fori_loop(0, NP2, _pair, 0)
        plsc.subcore_barrier()

        def _update(u, _):
            roff = s * RPT + u * CHUNK
            hoff = c * NPAD + roff
            pltpu.sync_copy(agg_sp.at[pl.ds(roff, CHUNK)], a_v)
            pltpu.sync_copy(x_hbm.at[pl.ds(hoff, CHUNK)], b_v)

            def _fma(i, _):
                r = i >> 2
                sl = pl.ds((i & 3) * 16, 16)
                a_v[r, sl] = a_v[r, sl] * (1.0 - ALPHA) + b_v[r, sl] * ALPHA
                return 0
            lax.fori_loop(0, CHUNK * 4, _fma, 0)
            pltpu.sync_copy(a_v, h_hbm.at[pl.ds(hoff, CHUNK)])
            return 0
        with jax.named_scope("ph_upd"):
            lax.fori_loop(0, UCH, _update, 0)
        plsc.subcore_barrier()
        return 0

    lax.fori_loop(0, K, _round, 0)


@functools.partial(
    pl.kernel,
    out_type=jax.ShapeDtypeStruct((2 * NPAD, DH), jnp.float32),
    mesh=plsc.VectorSubcoreMesh(core_axis_name="c", subcore_axis_name="s"),
    compiler_params=pltpu.CompilerParams(needs_layout_passes=False,
                                         use_tc_tiling_on_sc=False),
    scratch_types=[
        pltpu.VMEM((NCH, CHUNK), jnp.int32),
        pltpu.VMEM((EPT,), jnp.float32),
        pltpu.VMEM((CHUNK, DH), jnp.float32),
        pltpu.VMEM((CHUNK, DH), jnp.float32),
        pltpu.VMEM((CHUNK,), jnp.int32),
        pltpu.VMEM((CHUNK,), jnp.int32),
        pltpu.VMEM((CHUNK,), jnp.int32),
        pltpu.VMEM((CHUNK,), jnp.int32),
        pltpu.VMEM((CHUNK, DH), jnp.float32),
        pltpu.VMEM((CHUNK, DH), jnp.float32),
        pltpu.SemaphoreType.DMA,
        pltpu.SemaphoreType.DMA,
        pltpu.VMEM_SHARED((NPAD, DH), jnp.float32),
    ],
)
def _appnp_sc(pk_hbm, val_hbm, x_hbm, h_hbm,
              pk_t, val_t, r0, r1, cc0, rc0, cc1, rc1, a_v, b_v,
              gs0, gs1, agg_sp):
    _appnp_body(pk_hbm, val_hbm, x_hbm, h_hbm,
                pk_t, val_t, r0, r1, cc0, rc0, cc1, rc1, a_v, b_v,
                gs0, gs1, agg_sp)


N_EDGES_FIXED = 320000


def _threefry2x32_np(k1, k2, x0, x1):
    import numpy as np
    r0 = (13, 15, 26, 6)
    r1 = (17, 29, 16, 24)
    ks0 = np.uint32(k1)
    ks1 = np.uint32(k2)
    ks2 = ks0 ^ ks1 ^ np.uint32(0x1BD11BDA)
    with np.errstate(over="ignore"):
        x0 = (x0 + ks0).astype(np.uint32)
        x1 = (x1 + ks1).astype(np.uint32)

        def rounds(x0, x1, rots):
            for r in rots:
                x0 = (x0 + x1).astype(np.uint32)
                x1 = ((x1 << np.uint32(r))
                      | (x1 >> np.uint32(32 - r))).astype(np.uint32)
                x1 = x1 ^ x0
            return x0, x1

        for rots, a0, a1, ctr in ((r0, ks1, ks2, 1), (r1, ks2, ks0, 2),
                                  (r0, ks0, ks1, 3), (r1, ks1, ks2, 4),
                                  (r0, ks2, ks0, 5)):
            x0, x1 = rounds(x0, x1, rots)
            x0 = (x0 + a0).astype(np.uint32)
            x1 = (x1 + a1 + np.uint32(ctr)).astype(np.uint32)
    return x0, x1


def _build_mask_constants():
    import numpy as np
    k1, k2 = np.uint32(0), np.uint32(42)
    masks = []
    for i in range(K):
        f1, f2 = _threefry2x32_np(k1, k2, np.uint32(0), np.uint32(i))
        hi = np.zeros(N_EDGES_FIXED, np.uint32)
        lo = np.arange(N_EDGES_FIXED, dtype=np.uint32)
        b1, b2 = _threefry2x32_np(f1, f2, hi, lo)
        bits = b1 ^ b2
        fb = (bits >> np.uint32(9)) | np.uint32(0x3F800000)
        floats = fb.view(np.float32) - np.float32(1.0)
        masks.append(floats < np.float32(1.0 - DROPOUT_P))
    return np.stack(masks)


_MASKS = _build_mask_constants()


def kernel(x, edge_index, edge_weight):
    row = edge_index[0]
    col = edge_index[1]
    n_edges = edge_weight.shape[0]
    epad = E_PAD - n_edges

    scale = 1.0 / (1.0 - DROPOUT_P)
    masks = jnp.asarray(_MASKS)
    val_all = jnp.where(masks, edge_weight * scale, 0.0)
    val_all = jnp.pad(val_all, ((0, 0), (0, epad)))
    packed = (row.astype(jnp.int32) << 16) | col.astype(jnp.int32)
    pad_i = jnp.arange(epad, dtype=jnp.int32)
    pad_pk = ((N_NODES + pad_i % (NPAD - N_NODES)) << 16) | (pad_i % N_NODES)
    packed = jnp.concatenate([packed, pad_pk])

    xpad = jnp.pad(x, ((0, NPAD - N_NODES), (0, 0)))
    x_flat = jnp.concatenate([xpad[:, :DH], xpad[:, DH:]], axis=0)

    h = _appnp_sc(packed.reshape(NT, NCH, CHUNK),
                  val_all.reshape(K * NT, EPT), x_flat)
    return jnp.concatenate([h[:N_NODES], h[NPAD:NPAD + N_NODES]], axis=1)
